# Initial kernel scaffold; baseline (speedup 1.0000x reference)
import jax
import jax.numpy as jnp
from jax.experimental import pallas as pl


def kernel(ebds, adj_edge_index, adj_values):
    raise NotImplementedError("write your pallas kernel here")


# speedup vs baseline: 2.5585x; 2.5585x over previous
import functools

import jax
import jax.numpy as jnp
from jax import lax
from jax.experimental import pallas as pl
from jax.experimental.pallas import tpu as pltpu
from jax.experimental.pallas import tpu_sc as plsc

NN = 10000
D = 128
NE = 320000
NC, NS, L = 2, 16, 16
NW = NC * NS
CH = 128
NCH = 80
PER_W = NCH * CH
EPAD = NW * PER_W
RPS = 624
TAIL = NN - NS * RPS


def _spmm_sc(x, src3, dst3, val3):
    mesh = plsc.VectorSubcoreMesh(core_axis_name="c", subcore_axis_name="s")

    @functools.partial(
        pl.kernel,
        mesh=mesh,
        out_type=jax.ShapeDtypeStruct((NC, NN, D), jnp.float32),
        scratch_types=[
            pltpu.VMEM((NCH, CH), jnp.int32),
            pltpu.VMEM((NCH, CH), jnp.int32),
            pltpu.VMEM((NCH, CH), jnp.float32),
            pltpu.VMEM((CH, D), jnp.float32),
            pltpu.VMEM_SHARED((NN, D), jnp.float32),
            pltpu.SemaphoreType.DMA,
        ],
    )
    def k(x_hbm, src_hbm, dst_hbm, val_hbm, out_hbm,
          src_v, dst_v, val_s, rows_v, acc_sh, sem):
        c = lax.axis_index("c")
        s = lax.axis_index("s")
        w = c * NS + s

        pltpu.sync_copy(src_hbm.at[w], src_v)
        pltpu.sync_copy(dst_hbm.at[w], dst_v)
        pltpu.sync_copy(val_hbm.at[w], val_s)

        zero = jnp.zeros((L,), jnp.float32)

        @pl.loop(0, CH)
        def _(e):
            for j in range(D // L):
                rows_v[e, pl.ds(j * L, L)] = zero

        for t in range(6):
            pltpu.sync_copy(rows_v.at[pl.ds(0, 104)],
                            acc_sh.at[pl.ds(s * RPS + t * 104, 104)])

        @pl.when(s == NS - 1)
        def _():
            pltpu.sync_copy(rows_v.at[pl.ds(0, TAIL)],
                            acc_sh.at[pl.ds(NS * RPS, TAIL)])

        plsc.subcore_barrier()

        @pl.loop(0, NCH)
        def _(cidx):
            pltpu.async_copy(x_hbm.at[src_v.at[cidx]], rows_v, sem).wait()

            @pl.loop(0, CH // L)
            def _(g):
                vals16 = val_s[cidx, pl.ds(g * L, L)]

                @pl.loop(0, L)
                def _(l):
                    bidx = jnp.broadcast_to(l, (L,)).astype(jnp.int32)
                    v = vals16.at[bidx].get(mode="promise_in_bounds")
                    e = g * L + l
                    for j in range(D // L):
                        sl = pl.ds(j * L, L)
                        rows_v[e, sl] = rows_v[e, sl] * v

            pltpu.sync_copy(rows_v, acc_sh.at[dst_v.at[cidx]], add=True)

        plsc.subcore_barrier()
        for t in range(6):
            r0 = s * RPS + t * 104
            pltpu.sync_copy(acc_sh.at[pl.ds(r0, 104)],
                            out_hbm.at[c, pl.ds(r0, 104)])

        @pl.when(s == NS - 1)
        def _():
            pltpu.sync_copy(acc_sh.at[pl.ds(NS * RPS, TAIL)],
                            out_hbm.at[c, pl.ds(NS * RPS, TAIL)])

    return k(x, src3, dst3, val3)


def _combine_tc(p, tot):
    RB = 1000

    def body(p_ref, t_ref, y_ref, o_ref):
        y = p_ref[0] + p_ref[1]
        y_ref[...] = y
        o_ref[...] = t_ref[...] + y

    return pl.pallas_call(
        body,
        grid=(NN // RB,),
        in_specs=[pl.BlockSpec((2, RB, D), lambda i: (0, i, 0)),
                  pl.BlockSpec((RB, D), lambda i: (i, 0))],
        out_specs=[pl.BlockSpec((RB, D), lambda i: (i, 0)),
                   pl.BlockSpec((RB, D), lambda i: (i, 0))],
        out_shape=[jax.ShapeDtypeStruct((NN, D), jnp.float32),
                   jax.ShapeDtypeStruct((NN, D), jnp.float32)],
    )(p, tot)


def kernel(ebds, adj_edge_index, adj_values):
    pad = EPAD - NE
    src = jnp.concatenate([adj_edge_index[0],
                           jnp.zeros((pad,), jnp.int32)]).reshape(NW, NCH, CH)
    dst = jnp.concatenate([adj_edge_index[1],
                           jnp.zeros((pad,), jnp.int32)]).reshape(NW, NCH, CH)
    val = jnp.concatenate([adj_values,
                           jnp.zeros((pad,), jnp.float32)]).reshape(NW, NCH, CH)

    x = ebds
    total = ebds
    for _ in range(3):
        p = _spmm_sc(x, src, dst, val)
        x, total = _combine_tc(p, total)
    return total


# speedup vs baseline: 3.6691x; 1.4341x over previous
import functools

import jax
import jax.numpy as jnp
from jax import lax
from jax.experimental import pallas as pl
from jax.experimental.pallas import tpu as pltpu
from jax.experimental.pallas import tpu_sc as plsc

NN = 10000
D = 128
DH = 64
NE = 320000
NC, NS, L = 2, 16, 16
CH = 128
NCH = 160
W = 80
PER_W = NCH * CH
EPAD = NS * PER_W
NBUF = 4
RPS = 624
TAIL = NN - NS * RPS


def _spmm_sc(x, src3, dst3, val3):
    mesh = plsc.VectorSubcoreMesh(core_axis_name="c", subcore_axis_name="s")

    @functools.partial(
        pl.kernel,
        mesh=mesh,
        compiler_params=pltpu.CompilerParams(use_tc_tiling_on_sc=False),
        out_type=jax.ShapeDtypeStruct((NC, NN, DH), jnp.float32),
        scratch_types=[
            pltpu.VMEM((W, CH), jnp.int32),
            pltpu.VMEM((W, CH), jnp.int32),
            pltpu.VMEM((W, CH), jnp.float32),
            pltpu.VMEM((NBUF, CH, DH), jnp.float32),
            pltpu.VMEM_SHARED((NN, DH), jnp.float32),
            pltpu.SemaphoreType.DMA,
            pltpu.SemaphoreType.DMA,
            pltpu.SemaphoreType.DMA,
            pltpu.SemaphoreType.DMA,
            pltpu.SemaphoreType.DMA,
            pltpu.SemaphoreType.DMA,
            pltpu.SemaphoreType.DMA,
            pltpu.SemaphoreType.DMA,
        ],
    )
    def k(x_hbm, src_hbm, dst_hbm, val_hbm, out_hbm,
          src_v, dst_v, val_v, rows_v, acc_sh,
          gs0, gs1, gs2, gs3, ss0, ss1, ss2, ss3):
        gsem = [gs0, gs1, gs2, gs3]
        ssem = [ss0, ss1, ss2, ss3]
        c = lax.axis_index("c")
        s = lax.axis_index("s")
        xh = x_hbm.at[c]

        zero = jnp.zeros((L,), jnp.float32)

        @pl.loop(0, CH)
        def _(e):
            for j in range(DH // L):
                rows_v[0, e, pl.ds(j * L, L)] = zero

        for t in range(6):
            pltpu.sync_copy(rows_v.at[0, pl.ds(0, 104)],
                            acc_sh.at[pl.ds(s * RPS + t * 104, 104)])

        @pl.when(s == NS - 1)
        def _():
            pltpu.sync_copy(rows_v.at[0, pl.ds(0, TAIL)],
                            acc_sh.at[pl.ds(NS * RPS, TAIL)])

        plsc.subcore_barrier()

        def issue_gather(cidx, b):
            pltpu.async_copy(xh.at[src_v.at[cidx]], rows_v.at[b], gsem[b])

        def wait_gather(b):
            pltpu.make_async_copy(xh.at[pl.ds(0, CH)], rows_v.at[b],
                                  gsem[b]).wait()

        def issue_scatter(cidx, b):
            pltpu.async_copy(rows_v.at[b], acc_sh.at[dst_v.at[cidx]],
                             ssem[b], add=True)

        def wait_scatter(b):
            pltpu.make_async_copy(rows_v.at[b], acc_sh.at[pl.ds(0, CH)],
                                  ssem[b]).wait()

        def scale(cidx, b):
            @pl.loop(0, CH // L)
            def _(g):
                vals16 = val_v[cidx, pl.ds(g * L, L)]

                @pl.loop(0, L)
                def _(l):
                    bidx = jnp.broadcast_to(l, (L,)).astype(jnp.int32)
                    v = vals16.at[bidx].get(mode="promise_in_bounds")
                    e = g * L + l
                    for j in range(DH // L):
                        sl = pl.ds(j * L, L)
                        rows_v[b, e, sl] = rows_v[b, e, sl] * v

        for p in range(2):
            pltpu.sync_copy(src_hbm.at[s, pl.ds(p * W, W)], src_v)
            pltpu.sync_copy(dst_hbm.at[s, pl.ds(p * W, W)], dst_v)
            pltpu.sync_copy(val_hbm.at[s, pl.ds(p * W, W)], val_v)

            issue_gather(0, 0)
            issue_gather(1, 1)

            @pl.loop(0, W // 4)
            def _(k_):
                for b in range(4):
                    cidx = k_ * 4 + b
                    bn = (b + 2) % 4
                    if b >= 2:
                        wait_scatter(bn)
                    else:
                        @pl.when(k_ > 0)
                        def _():
                            wait_scatter(bn)
                    if b <= 1:
                        issue_gather(cidx + 2, bn)
                    else:
                        @pl.when(k_ < W // 4 - 1)
                        def _():
                            issue_gather(cidx + 2, bn)
                    wait_gather(b)
                    scale(cidx, b)
                    issue_scatter(cidx, b)

            wait_scatter(2)
            wait_scatter(3)

        plsc.subcore_barrier()
        for t in range(6):
            r0 = s * RPS + t * 104
            pltpu.sync_copy(acc_sh.at[pl.ds(r0, 104)],
                            out_hbm.at[c, pl.ds(r0, 104)])

        @pl.when(s == NS - 1)
        def _():
            pltpu.sync_copy(acc_sh.at[pl.ds(NS * RPS, TAIL)],
                            out_hbm.at[c, pl.ds(NS * RPS, TAIL)])

    return k(x, src3, dst3, val3)


def _add_tc(tot, y):
    RB = 2000
    t2 = tot.reshape(NC * NN, DH)
    y2 = y.reshape(NC * NN, DH)

    def body(t_ref, y_ref, o_ref):
        o_ref[...] = t_ref[...] + y_ref[...]

    out = pl.pallas_call(
        body,
        grid=(NC * NN // RB,),
        in_specs=[pl.BlockSpec((RB, DH), lambda i: (i, 0)),
                  pl.BlockSpec((RB, DH), lambda i: (i, 0))],
        out_specs=pl.BlockSpec((RB, DH), lambda i: (i, 0)),
        out_shape=jax.ShapeDtypeStruct((NC * NN, DH), jnp.float32),
    )(t2, y2)
    return out.reshape(NC, NN, DH)


def kernel(ebds, adj_edge_index, adj_values):
    pad = EPAD - NE
    src = jnp.concatenate([adj_edge_index[0],
                           jnp.zeros((pad,), jnp.int32)]).reshape(NS, NCH, CH)
    dst = jnp.concatenate([adj_edge_index[1],
                           jnp.zeros((pad,), jnp.int32)]).reshape(NS, NCH, CH)
    val = jnp.concatenate([adj_values,
                           jnp.zeros((pad,), jnp.float32)]).reshape(NS, NCH, CH)

    x = ebds.reshape(NN, NC, DH).transpose(1, 0, 2)
    total = x
    for _ in range(3):
        x = _spmm_sc(x, src, dst, val)
        total = _add_tc(total, x)
    return total.transpose(1, 0, 2).reshape(NN, D)


# speedup vs baseline: 4.3543x; 1.1867x over previous
import functools

import jax
import jax.numpy as jnp
from jax import lax
from jax.experimental import pallas as pl
from jax.experimental.pallas import tpu as pltpu
from jax.experimental.pallas import tpu_sc as plsc

NN = 10000
D = 128
DH = 64
NE = 320000
NC, NS, L = 2, 16, 16
CH = 128
NCH = 160
W = 40
PER_W = NCH * CH
EPAD = NS * PER_W
NBUF = 8
LA = 4
RPS = 624
TAIL = NN - NS * RPS


def _spmm_sc(x, src3, dst3, val3):
    mesh = plsc.VectorSubcoreMesh(core_axis_name="c", subcore_axis_name="s")

    @functools.partial(
        pl.kernel,
        mesh=mesh,
        compiler_params=pltpu.CompilerParams(use_tc_tiling_on_sc=False),
        out_type=jax.ShapeDtypeStruct((NC, NN, DH), jnp.float32),
        scratch_types=[
            pltpu.VMEM((W, CH), jnp.int32),
            pltpu.VMEM((W, CH), jnp.int32),
            pltpu.VMEM((W, CH), jnp.float32),
            pltpu.VMEM((NBUF, CH, DH), jnp.float32),
            pltpu.VMEM_SHARED((NN, DH), jnp.float32),
        ] + [pltpu.SemaphoreType.DMA] * (2 * NBUF),
    )
    def k(x_hbm, src_hbm, dst_hbm, val_hbm, out_hbm,
          src_v, dst_v, val_v, rows_v, acc_sh, *sems):
        gsem = list(sems[:NBUF])
        ssem = list(sems[NBUF:])
        c = lax.axis_index("c")
        s = lax.axis_index("s")
        xh = x_hbm.at[c]

        zero = jnp.zeros((L,), jnp.float32)

        @pl.loop(0, CH)
        def _(e):
            for j in range(DH // L):
                rows_v[0, e, pl.ds(j * L, L)] = zero

        for t in range(6):
            pltpu.sync_copy(rows_v.at[0, pl.ds(0, 104)],
                            acc_sh.at[pl.ds(s * RPS + t * 104, 104)])

        @pl.when(s == NS - 1)
        def _():
            pltpu.sync_copy(rows_v.at[0, pl.ds(0, TAIL)],
                            acc_sh.at[pl.ds(NS * RPS, TAIL)])

        plsc.subcore_barrier()

        def issue_gather(cidx, b):
            pltpu.async_copy(xh.at[src_v.at[cidx]], rows_v.at[b], gsem[b])

        def wait_gather(b):
            pltpu.make_async_copy(xh.at[pl.ds(0, CH)], rows_v.at[b],
                                  gsem[b]).wait()

        def issue_scatter(cidx, b):
            pltpu.async_copy(rows_v.at[b], acc_sh.at[dst_v.at[cidx]],
                             ssem[b], add=True)

        def wait_scatter(b):
            pltpu.make_async_copy(rows_v.at[b], acc_sh.at[pl.ds(0, CH)],
                                  ssem[b]).wait()

        def scale(cidx, b):
            @pl.loop(0, CH // L)
            def _(g):
                vals16 = val_v[cidx, pl.ds(g * L, L)]
                for l in range(L):
                    bidx = jnp.full((L,), l, jnp.int32)
                    v = vals16.at[bidx].get(mode="promise_in_bounds")
                    e = g * L + l
                    for j in range(DH // L):
                        sl = pl.ds(j * L, L)
                        rows_v[b, e, sl] = rows_v[b, e, sl] * v

        @pl.loop(0, NCH // W)
        def _(p):
            pltpu.sync_copy(src_hbm.at[s, pl.ds(p * W, W)], src_v)
            pltpu.sync_copy(dst_hbm.at[s, pl.ds(p * W, W)], dst_v)
            pltpu.sync_copy(val_hbm.at[s, pl.ds(p * W, W)], val_v)

            for b in range(LA):
                issue_gather(b, b)

            @pl.loop(0, W // NBUF)
            def _(k_):
                for b in range(NBUF):
                    cidx = k_ * NBUF + b
                    bn = (b + LA) % NBUF
                    if b >= LA:
                        wait_scatter(bn)
                    else:
                        @pl.when(k_ > 0)
                        def _():
                            wait_scatter(bn)
                    if b < NBUF - LA:
                        issue_gather(cidx + LA, bn)
                    else:
                        @pl.when(k_ < W // NBUF - 1)
                        def _():
                            issue_gather(cidx + LA, bn)
                    wait_gather(b)
                    scale(cidx, b)
                    issue_scatter(cidx, b)

            for b in range(LA, NBUF):
                wait_scatter(b)

        plsc.subcore_barrier()
        for t in range(6):
            r0 = s * RPS + t * 104
            pltpu.sync_copy(acc_sh.at[pl.ds(r0, 104)],
                            out_hbm.at[c, pl.ds(r0, 104)])

        @pl.when(s == NS - 1)
        def _():
            pltpu.sync_copy(acc_sh.at[pl.ds(NS * RPS, TAIL)],
                            out_hbm.at[c, pl.ds(NS * RPS, TAIL)])

    return k(x, src3, dst3, val3)


def _add_tc(tot, y):
    RB = 2000
    t2 = tot.reshape(NC * NN, DH)
    y2 = y.reshape(NC * NN, DH)

    def body(t_ref, y_ref, o_ref):
        o_ref[...] = t_ref[...] + y_ref[...]

    out = pl.pallas_call(
        body,
        grid=(NC * NN // RB,),
        in_specs=[pl.BlockSpec((RB, DH), lambda i: (i, 0)),
                  pl.BlockSpec((RB, DH), lambda i: (i, 0))],
        out_specs=pl.BlockSpec((RB, DH), lambda i: (i, 0)),
        out_shape=jax.ShapeDtypeStruct((NC * NN, DH), jnp.float32),
    )(t2, y2)
    return out.reshape(NC, NN, DH)


def kernel(ebds, adj_edge_index, adj_values):
    pad = EPAD - NE
    src = jnp.concatenate([adj_edge_index[0],
                           jnp.zeros((pad,), jnp.int32)]).reshape(NS, NCH, CH)
    dst = jnp.concatenate([adj_edge_index[1],
                           jnp.zeros((pad,), jnp.int32)]).reshape(NS, NCH, CH)
    val = jnp.concatenate([adj_values,
                           jnp.zeros((pad,), jnp.float32)]).reshape(NS, NCH, CH)

    x = ebds.reshape(NN, NC, DH).transpose(1, 0, 2)
    total = x
    for _ in range(3):
        x = _spmm_sc(x, src, dst, val)
        total = _add_tc(total, x)
    return total.transpose(1, 0, 2).reshape(NN, D)


# speedup vs baseline: 5.2745x; 1.2113x over previous
import functools

import jax
import jax.numpy as jnp
from jax import lax
from jax.experimental import pallas as pl
from jax.experimental.pallas import tpu as pltpu
from jax.experimental.pallas import tpu_sc as plsc

NN = 10000
D = 128
DH = 64
NE = 320000
NC, NS, L = 2, 16, 16
CH = 128
NCH = 160
W = 20
PER_W = NCH * CH
EPAD = NS * PER_W
NBUF = 4
LA = 2
ZR = 64
RPS = 640
NNP = NS * RPS
NB = RPS // CH


def _lightgcn_sc(x, src3, dst3, val3):
    mesh = plsc.VectorSubcoreMesh(core_axis_name="c", subcore_axis_name="s")

    @functools.partial(
        pl.kernel,
        mesh=mesh,
        compiler_params=pltpu.CompilerParams(use_tc_tiling_on_sc=False),
        out_type=jax.ShapeDtypeStruct((3, NC, NN, DH), jnp.float32),
        scratch_types=[
            pltpu.VMEM((W, CH), jnp.int32),
            pltpu.VMEM((W, CH), jnp.int32),
            pltpu.VMEM((W, CH), jnp.float32),
            pltpu.VMEM((NBUF, CH, DH), jnp.float32),
            pltpu.VMEM((ZR, DH), jnp.float32),
            pltpu.VMEM_SHARED((NNP, DH), jnp.float32),
            pltpu.VMEM_SHARED((NNP, DH), jnp.float32),
        ] + [pltpu.SemaphoreType.DMA] * (2 * NBUF),
    )
    def k(x_hbm, src_hbm, dst_hbm, val_hbm, out_hbm,
          src_v, dst_v, val_v, rows_v, zero_v, a_sh, b_sh,
          *sems):
        gsem = list(sems[:NBUF])
        ssem = list(sems[NBUF:])
        c = lax.axis_index("c")
        s = lax.axis_index("s")
        r0 = s * RPS
        xh = x_hbm.at[c]

        z16 = jnp.zeros((L,), jnp.float32)

        @pl.loop(0, ZR)
        def _(e):
            for j in range(DH // L):
                zero_v[e, pl.ds(j * L, L)] = z16

        pltpu.sync_copy(xh.at[pl.ds(r0, 384)], a_sh.at[pl.ds(r0, 384)])

        @pl.when(s < NS - 1)
        def _():
            pltpu.sync_copy(xh.at[pl.ds(r0 + 384, RPS - 384)],
                            a_sh.at[pl.ds(r0 + 384, RPS - 384)])

        @pl.when(s == NS - 1)
        def _():
            pltpu.sync_copy(xh.at[pl.ds(r0 + 384, 16)],
                            a_sh.at[pl.ds(r0 + 384, 16)])

        for t in range(RPS // ZR):
            pltpu.sync_copy(zero_v, b_sh.at[pl.ds(r0 + t * ZR, ZR)])

        plsc.subcore_barrier()

        def edge_loop(src_ref, dst_ref):
            def issue_gather(cidx, b):
                pltpu.async_copy(src_ref.at[src_v.at[cidx]], rows_v.at[b],
                                 gsem[b])

            def wait_gather(b):
                pltpu.make_async_copy(src_ref.at[pl.ds(0, CH)], rows_v.at[b],
                                      gsem[b]).wait()

            def issue_scatter(cidx, b):
                pltpu.async_copy(rows_v.at[b], dst_ref.at[dst_v.at[cidx]],
                                 ssem[b], add=True)

            def wait_scatter(b):
                pltpu.make_async_copy(rows_v.at[b], dst_ref.at[pl.ds(0, CH)],
                                      ssem[b]).wait()

            def scale(cidx, b):
                @pl.loop(0, CH // L)
                def _(g):
                    vals16 = val_v[cidx, pl.ds(g * L, L)]
                    for l in range(L):
                        bidx = jnp.full((L,), l, jnp.int32)
                        v = vals16.at[bidx].get(mode="promise_in_bounds")
                        e = g * L + l
                        for j in range(DH // L):
                            sl = pl.ds(j * L, L)
                            rows_v[b, e, sl] = rows_v[b, e, sl] * v

            @pl.loop(0, NCH // W)
            def _(p):
                pltpu.sync_copy(src_hbm.at[s, pl.ds(p * W, W)], src_v)
                pltpu.sync_copy(dst_hbm.at[s, pl.ds(p * W, W)], dst_v)
                pltpu.sync_copy(val_hbm.at[s, pl.ds(p * W, W)], val_v)

                for b in range(LA):
                    issue_gather(b, b)

                @pl.loop(0, W // NBUF)
                def _(k_):
                    for b in range(NBUF):
                        cidx = k_ * NBUF + b
                        bn = (b + LA) % NBUF
                        if b >= LA:
                            wait_scatter(bn)
                        else:
                            @pl.when(k_ > 0)
                            def _():
                                wait_scatter(bn)
                        if b < NBUF - LA:
                            issue_gather(cidx + LA, bn)
                        else:
                            @pl.when(k_ < W // NBUF - 1)
                            def _():
                                issue_gather(cidx + LA, bn)
                        wait_gather(b)
                        scale(cidx, b)
                        issue_scatter(cidx, b)

                for b in range(LA, NBUF):
                    wait_scatter(b)

        def write_layer(dref, li):
            pltpu.sync_copy(dref.at[pl.ds(r0, 384)],
                            out_hbm.at[li, c, pl.ds(r0, 384)])

            @pl.when(s < NS - 1)
            def _():
                pltpu.sync_copy(dref.at[pl.ds(r0 + 384, RPS - 384)],
                                out_hbm.at[li, c, pl.ds(r0 + 384, RPS - 384)])

            @pl.when(s == NS - 1)
            def _():
                pltpu.sync_copy(dref.at[pl.ds(r0 + 384, 16)],
                                out_hbm.at[li, c, pl.ds(r0 + 384, 16)])

        def zero_slice(dref):
            for t in range(RPS // ZR):
                pltpu.sync_copy(zero_v, dref.at[pl.ds(r0 + t * ZR, ZR)])

        edge_loop(a_sh, b_sh)
        plsc.subcore_barrier()
        write_layer(b_sh, 0)
        zero_slice(a_sh)
        plsc.subcore_barrier()
        edge_loop(b_sh, a_sh)
        plsc.subcore_barrier()
        write_layer(a_sh, 1)
        zero_slice(b_sh)
        plsc.subcore_barrier()
        edge_loop(a_sh, b_sh)
        plsc.subcore_barrier()
        write_layer(b_sh, 2)

    return k(x, src3, dst3, val3)


def _sum4_tc(x, layers):
    RB = 2000
    x2 = x.reshape(NC * NN, DH)
    l2 = layers.reshape(3, NC * NN, DH)

    def body(x_ref, l_ref, o_ref):
        o_ref[...] = (x_ref[...] + l_ref[0] + l_ref[1] + l_ref[2])

    out = pl.pallas_call(
        body,
        grid=(NC * NN // RB,),
        in_specs=[pl.BlockSpec((RB, DH), lambda i: (i, 0)),
                  pl.BlockSpec((3, RB, DH), lambda i: (0, i, 0))],
        out_specs=pl.BlockSpec((RB, DH), lambda i: (i, 0)),
        out_shape=jax.ShapeDtypeStruct((NC * NN, DH), jnp.float32),
    )(x2, l2)
    return out.reshape(NC, NN, DH)


def kernel(ebds, adj_edge_index, adj_values):
    pad = EPAD - NE
    src = jnp.concatenate([adj_edge_index[0],
                           jnp.zeros((pad,), jnp.int32)]).reshape(NS, NCH, CH)
    dst = jnp.concatenate([adj_edge_index[1],
                           jnp.zeros((pad,), jnp.int32)]).reshape(NS, NCH, CH)
    val = jnp.concatenate([adj_values,
                           jnp.zeros((pad,), jnp.float32)]).reshape(NS, NCH, CH)

    x = ebds.reshape(NN, NC, DH).transpose(1, 0, 2)
    layers = _lightgcn_sc(x, src, dst, val)
    total = _sum4_tc(x, layers)
    return total.transpose(1, 0, 2).reshape(NN, D)


# speedup vs baseline: 5.9915x; 1.1359x over previous
import functools

import jax
import jax.numpy as jnp
from jax import lax
from jax.experimental import pallas as pl
from jax.experimental.pallas import tpu as pltpu
from jax.experimental.pallas import tpu_sc as plsc

NN = 10000
D = 128
DH = 64
NE = 320000
NC, NS, L = 2, 16, 16
CH = 64
NCH = 320
W = 20
PER_W = NCH * CH
EPAD = NS * PER_W
NBUF = 10
LA = 5
ZR = 32
RPS = 640
NNP = NS * RPS
NB = RPS // CH


def _lightgcn_sc(x, src3, dst3, val3):
    mesh = plsc.VectorSubcoreMesh(core_axis_name="c", subcore_axis_name="s")

    @functools.partial(
        pl.kernel,
        mesh=mesh,
        compiler_params=pltpu.CompilerParams(use_tc_tiling_on_sc=False),
        out_type=jax.ShapeDtypeStruct((3, NC, NN, DH), jnp.float32),
        scratch_types=[
            pltpu.VMEM((W, CH), jnp.int32),
            pltpu.VMEM((W, CH), jnp.int32),
            pltpu.VMEM((W, CH), jnp.float32),
            pltpu.VMEM((NBUF, CH, DH), jnp.float32),
            pltpu.VMEM((ZR, DH), jnp.float32),
            pltpu.VMEM_SHARED((NNP, DH), jnp.float32),
            pltpu.VMEM_SHARED((NNP, DH), jnp.float32),
        ] + [pltpu.SemaphoreType.DMA] * (2 * NBUF),
    )
    def k(x_hbm, src_hbm, dst_hbm, val_hbm, out_hbm,
          src_v, dst_v, val_v, rows_v, zero_v, a_sh, b_sh,
          *sems):
        gsem = list(sems[:NBUF])
        ssem = list(sems[NBUF:])
        c = lax.axis_index("c")
        s = lax.axis_index("s")
        r0 = s * RPS
        xh = x_hbm.at[c]

        z16 = jnp.zeros((L,), jnp.float32)

        @pl.loop(0, ZR)
        def _(e):
            for j in range(DH // L):
                zero_v[e, pl.ds(j * L, L)] = z16

        pltpu.sync_copy(xh.at[pl.ds(r0, 384)], a_sh.at[pl.ds(r0, 384)])

        @pl.when(s < NS - 1)
        def _():
            pltpu.sync_copy(xh.at[pl.ds(r0 + 384, RPS - 384)],
                            a_sh.at[pl.ds(r0 + 384, RPS - 384)])

        @pl.when(s == NS - 1)
        def _():
            pltpu.sync_copy(xh.at[pl.ds(r0 + 384, 16)],
                            a_sh.at[pl.ds(r0 + 384, 16)])

        for t in range(RPS // ZR):
            pltpu.sync_copy(zero_v, b_sh.at[pl.ds(r0 + t * ZR, ZR)])

        plsc.subcore_barrier()

        def edge_loop(src_ref, dst_ref):
            def issue_gather(cidx, b):
                pltpu.async_copy(src_ref.at[src_v.at[cidx]], rows_v.at[b],
                                 gsem[b])

            def wait_gather(b):
                pltpu.make_async_copy(src_ref.at[pl.ds(0, CH)], rows_v.at[b],
                                      gsem[b]).wait()

            def issue_scatter(cidx, b):
                pltpu.async_copy(rows_v.at[b], dst_ref.at[dst_v.at[cidx]],
                                 ssem[b], add=True)

            def wait_scatter(b):
                pltpu.make_async_copy(rows_v.at[b], dst_ref.at[pl.ds(0, CH)],
                                      ssem[b]).wait()

            def scale(cidx, b):
                @pl.loop(0, CH // L)
                def _(g):
                    vals16 = val_v[cidx, pl.ds(g * L, L)]
                    for l in range(L):
                        bidx = jnp.full((L,), l, jnp.int32)
                        v = vals16.at[bidx].get(mode="promise_in_bounds")
                        e = g * L + l
                        for j in range(DH // L):
                            sl = pl.ds(j * L, L)
                            rows_v[b, e, sl] = rows_v[b, e, sl] * v

            @pl.loop(0, NCH // W)
            def _(p):
                pltpu.sync_copy(src_hbm.at[s, pl.ds(p * W, W)], src_v)
                pltpu.sync_copy(dst_hbm.at[s, pl.ds(p * W, W)], dst_v)
                pltpu.sync_copy(val_hbm.at[s, pl.ds(p * W, W)], val_v)

                for b in range(LA):
                    issue_gather(b, b)

                @pl.loop(0, W // NBUF)
                def _(k_):
                    for b in range(NBUF):
                        cidx = k_ * NBUF + b
                        bn = (b + LA) % NBUF
                        if b >= LA:
                            wait_scatter(bn)
                        else:
                            @pl.when(k_ > 0)
                            def _():
                                wait_scatter(bn)
                        if b < NBUF - LA:
                            issue_gather(cidx + LA, bn)
                        else:
                            @pl.when(k_ < W // NBUF - 1)
                            def _():
                                issue_gather(cidx + LA, bn)
                        wait_gather(b)
                        scale(cidx, b)
                        issue_scatter(cidx, b)

                for b in range(LA, NBUF):
                    wait_scatter(b)

        def write_layer(dref, li):
            pltpu.sync_copy(dref.at[pl.ds(r0, 384)],
                            out_hbm.at[li, c, pl.ds(r0, 384)])

            @pl.when(s < NS - 1)
            def _():
                pltpu.sync_copy(dref.at[pl.ds(r0 + 384, RPS - 384)],
                                out_hbm.at[li, c, pl.ds(r0 + 384, RPS - 384)])

            @pl.when(s == NS - 1)
            def _():
                pltpu.sync_copy(dref.at[pl.ds(r0 + 384, 16)],
                                out_hbm.at[li, c, pl.ds(r0 + 384, 16)])

        def zero_slice(dref):
            for t in range(RPS // ZR):
                pltpu.sync_copy(zero_v, dref.at[pl.ds(r0 + t * ZR, ZR)])

        edge_loop(a_sh, b_sh)
        plsc.subcore_barrier()
        write_layer(b_sh, 0)
        zero_slice(a_sh)
        plsc.subcore_barrier()
        edge_loop(b_sh, a_sh)
        plsc.subcore_barrier()
        write_layer(a_sh, 1)
        zero_slice(b_sh)
        plsc.subcore_barrier()
        edge_loop(a_sh, b_sh)
        plsc.subcore_barrier()
        write_layer(b_sh, 2)

    return k(x, src3, dst3, val3)


def _sum4_tc(x, layers):
    RB = 2000
    x2 = x.reshape(NC * NN, DH)
    l2 = layers.reshape(3, NC * NN, DH)

    def body(x_ref, l_ref, o_ref):
        o_ref[...] = (x_ref[...] + l_ref[0] + l_ref[1] + l_ref[2])

    out = pl.pallas_call(
        body,
        grid=(NC * NN // RB,),
        in_specs=[pl.BlockSpec((RB, DH), lambda i: (i, 0)),
                  pl.BlockSpec((3, RB, DH), lambda i: (0, i, 0))],
        out_specs=pl.BlockSpec((RB, DH), lambda i: (i, 0)),
        out_shape=jax.ShapeDtypeStruct((NC * NN, DH), jnp.float32),
    )(x2, l2)
    return out.reshape(NC, NN, DH)


def kernel(ebds, adj_edge_index, adj_values):
    pad = EPAD - NE
    src = jnp.concatenate([adj_edge_index[0],
                           jnp.zeros((pad,), jnp.int32)]).reshape(NS, NCH, CH)
    dst = jnp.concatenate([adj_edge_index[1],
                           jnp.zeros((pad,), jnp.int32)]).reshape(NS, NCH, CH)
    val = jnp.concatenate([adj_values,
                           jnp.zeros((pad,), jnp.float32)]).reshape(NS, NCH, CH)

    x = ebds.reshape(NN, NC, DH).transpose(1, 0, 2)
    layers = _lightgcn_sc(x, src, dst, val)
    total = _sum4_tc(x, layers)
    return total.transpose(1, 0, 2).reshape(NN, D)


# speedup vs baseline: 6.2216x; 1.0384x over previous
import functools

import jax
import jax.numpy as jnp
from jax import lax
from jax.experimental import pallas as pl
from jax.experimental.pallas import tpu as pltpu
from jax.experimental.pallas import tpu_sc as plsc

NN = 10000
D = 128
DH = 64
NE = 320000
NC, NS, L = 2, 16, 16
CH = 64
NCH = 320
W = 20
PER_W = NCH * CH
EPAD = NS * PER_W
NBUF = 10
LA = 5
ZR = 32
RPS = 640
NNP = NS * RPS
NB = RPS // CH


def _lightgcn_sc(x, src3, dst3, val3):
    mesh = plsc.VectorSubcoreMesh(core_axis_name="c", subcore_axis_name="s")

    @functools.partial(
        pl.kernel,
        mesh=mesh,
        compiler_params=pltpu.CompilerParams(use_tc_tiling_on_sc=False),
        out_type=jax.ShapeDtypeStruct((3, NC, NN, DH), jnp.float32),
        scratch_types=[
            pltpu.VMEM((W, CH), jnp.int32),
            pltpu.VMEM((W, CH), jnp.int32),
            pltpu.VMEM((W, CH), jnp.float32),
            pltpu.VMEM((NBUF, CH, DH), jnp.float32),
            pltpu.VMEM((ZR, DH), jnp.float32),
            pltpu.VMEM_SHARED((NNP, DH), jnp.float32),
            pltpu.VMEM_SHARED((NNP, DH), jnp.float32),
        ] + [pltpu.SemaphoreType.DMA] * (2 * NBUF + 1),
    )
    def k(x_hbm, src_hbm, dst_hbm, val_hbm, out_hbm,
          src_v, dst_v, val_v, rows_v, zero_v, a_sh, b_sh,
          *sems):
        gsem = list(sems[:NBUF])
        ssem = list(sems[NBUF:2 * NBUF])
        isem = sems[2 * NBUF]
        c = lax.axis_index("c")
        s = lax.axis_index("s")
        r0 = s * RPS
        xh = x_hbm.at[c]

        z16 = jnp.zeros((L,), jnp.float32)

        @pl.loop(0, ZR)
        def _(e):
            for j in range(DH // L):
                zero_v[e, pl.ds(j * L, L)] = z16

        pltpu.sync_copy(xh.at[pl.ds(r0, 384)], a_sh.at[pl.ds(r0, 384)])

        @pl.when(s < NS - 1)
        def _():
            pltpu.sync_copy(xh.at[pl.ds(r0 + 384, RPS - 384)],
                            a_sh.at[pl.ds(r0 + 384, RPS - 384)])

        @pl.when(s == NS - 1)
        def _():
            pltpu.sync_copy(xh.at[pl.ds(r0 + 384, 16)],
                            a_sh.at[pl.ds(r0 + 384, 16)])

        for t in range(RPS // ZR):
            pltpu.sync_copy(zero_v, b_sh.at[pl.ds(r0 + t * ZR, ZR)])

        plsc.subcore_barrier()

        def edge_loop(src_ref, dst_ref):
            def issue_gather(cidx, b):
                pltpu.async_copy(src_ref.at[src_v.at[cidx]], rows_v.at[b],
                                 gsem[b])

            def wait_gather(b):
                pltpu.make_async_copy(src_ref.at[pl.ds(0, CH)], rows_v.at[b],
                                      gsem[b]).wait()

            def issue_scatter(cidx, b):
                pltpu.async_copy(rows_v.at[b], dst_ref.at[dst_v.at[cidx]],
                                 ssem[b], add=True)

            def wait_scatter(b):
                pltpu.make_async_copy(rows_v.at[b], dst_ref.at[pl.ds(0, CH)],
                                      ssem[b]).wait()

            def scale(cidx, b):
                @pl.loop(0, CH // L)
                def _(g):
                    vals16 = val_v[cidx, pl.ds(g * L, L)]
                    for l in range(L):
                        bidx = jnp.full((L,), l, jnp.int32)
                        v = vals16.at[bidx].get(mode="promise_in_bounds")
                        e = g * L + l
                        for j in range(DH // L):
                            sl = pl.ds(j * L, L)
                            rows_v[b, e, sl] = rows_v[b, e, sl] * v

            @pl.loop(0, NCH // W)
            def _(p):
                pltpu.async_copy(src_hbm.at[s, pl.ds(p * W, W)], src_v, isem)
                pltpu.async_copy(dst_hbm.at[s, pl.ds(p * W, W)], dst_v, isem)
                pltpu.async_copy(val_hbm.at[s, pl.ds(p * W, W)], val_v, isem)
                pltpu.make_async_copy(src_hbm.at[s, pl.ds(0, W)], src_v,
                                      isem).wait()
                pltpu.make_async_copy(dst_hbm.at[s, pl.ds(0, W)], dst_v,
                                      isem).wait()
                pltpu.make_async_copy(val_hbm.at[s, pl.ds(0, W)], val_v,
                                      isem).wait()

                for b in range(LA):
                    issue_gather(b, b)

                @pl.loop(0, W // NBUF)
                def _(k_):
                    for b in range(NBUF):
                        cidx = k_ * NBUF + b
                        bn = (b + LA) % NBUF
                        if b >= LA:
                            wait_scatter(bn)
                        else:
                            @pl.when(k_ > 0)
                            def _():
                                wait_scatter(bn)
                        if b < NBUF - LA:
                            issue_gather(cidx + LA, bn)
                        else:
                            @pl.when(k_ < W // NBUF - 1)
                            def _():
                                issue_gather(cidx + LA, bn)
                        wait_gather(b)
                        scale(cidx, b)
                        issue_scatter(cidx, b)

                for b in range(LA, NBUF):
                    wait_scatter(b)

        def write_layer(dref, li):
            pltpu.sync_copy(dref.at[pl.ds(r0, 384)],
                            out_hbm.at[li, c, pl.ds(r0, 384)])

            @pl.when(s < NS - 1)
            def _():
                pltpu.sync_copy(dref.at[pl.ds(r0 + 384, RPS - 384)],
                                out_hbm.at[li, c, pl.ds(r0 + 384, RPS - 384)])

            @pl.when(s == NS - 1)
            def _():
                pltpu.sync_copy(dref.at[pl.ds(r0 + 384, 16)],
                                out_hbm.at[li, c, pl.ds(r0 + 384, 16)])

        def zero_slice(dref):
            for t in range(RPS // ZR):
                pltpu.sync_copy(zero_v, dref.at[pl.ds(r0 + t * ZR, ZR)])

        edge_loop(a_sh, b_sh)
        plsc.subcore_barrier()
        write_layer(b_sh, 0)
        zero_slice(a_sh)
        plsc.subcore_barrier()
        edge_loop(b_sh, a_sh)
        plsc.subcore_barrier()
        write_layer(a_sh, 1)
        zero_slice(b_sh)
        plsc.subcore_barrier()
        edge_loop(a_sh, b_sh)
        plsc.subcore_barrier()
        write_layer(b_sh, 2)

    return k(x, src3, dst3, val3)


def _sum4_tc(x, layers):
    RB = 2000
    x2 = x.reshape(NC * NN, DH)
    l2 = layers.reshape(3, NC * NN, DH)

    def body(x_ref, l_ref, o_ref):
        o_ref[...] = (x_ref[...] + l_ref[0] + l_ref[1] + l_ref[2])

    out = pl.pallas_call(
        body,
        grid=(NC * NN // RB,),
        in_specs=[pl.BlockSpec((RB, DH), lambda i: (i, 0)),
                  pl.BlockSpec((3, RB, DH), lambda i: (0, i, 0))],
        out_specs=pl.BlockSpec((RB, DH), lambda i: (i, 0)),
        out_shape=jax.ShapeDtypeStruct((NC * NN, DH), jnp.float32),
    )(x2, l2)
    return out.reshape(NC, NN, DH)


def kernel(ebds, adj_edge_index, adj_values):
    pad = EPAD - NE
    src = jnp.concatenate([adj_edge_index[0],
                           jnp.zeros((pad,), jnp.int32)]).reshape(NS, NCH, CH)
    dst = jnp.concatenate([adj_edge_index[1],
                           jnp.zeros((pad,), jnp.int32)]).reshape(NS, NCH, CH)
    val = jnp.concatenate([adj_values,
                           jnp.zeros((pad,), jnp.float32)]).reshape(NS, NCH, CH)

    x = ebds.reshape(NN, NC, DH).transpose(1, 0, 2)
    layers = _lightgcn_sc(x, src, dst, val)
    total = _sum4_tc(x, layers)
    return total.transpose(1, 0, 2).reshape(NN, D)
